# baseline (speedup 1.0000x reference)
import functools

import jax
import jax.numpy as jnp
from jax import lax
from jax.experimental import pallas as pl
from jax.experimental.pallas import tpu as pltpu
from jax.experimental.pallas import tpu_sc as plsc

_NL = 16
_RADIX = 256
_DMASK = _RADIX - 1
_SHIFTS = (0, 8, 16, 24)
_NR = 2
_NSEG = 8
_UNROLL = 4


def _rows_sort_and_label(rows, S):
    NV = S // _NL
    SEGV = NV // _NSEG
    SEGSH = SEGV.bit_length() - 1
    assert SEGV == 1 << SEGSH and NV % _NSEG == 0
    NHC = _NSEG * _RADIX // _NL
    NC = _RADIX // _NL

    @plsc.parallel_loop(0, NV, unroll=_UNROLL)
    def key_init(j):
        sl = pl.ds(j * _NL, _NL)
        for (vbuf, key_a, pay_a, *_r) in rows:
            b = plsc.bitcast(vbuf[sl], jnp.int32)
            flip = jnp.where(b < 0, jnp.int32(-1), jnp.int32(-(2 ** 31)))
            key_a[sl] = b ^ flip
            pay_a[sl] = lax.iota(jnp.int32, _NL) + j * _NL

    def _one_pass(ios, sh):

        @plsc.parallel_loop(0, NHC, unroll=_UNROLL)
        def zero_body(c, ios=ios):
            sl = pl.ds(c * _NL, _NL)
            for io in ios:
                io[4][sl] = jnp.zeros((_NL,), jnp.int32)

        shv = sh + jnp.zeros((_NL,), jnp.int32)

        @plsc.parallel_loop(0, NV, unroll=_UNROLL)
        def hist_body(j, ios=ios, shv=shv):
            sl = pl.ds(j * _NL, _NL)
            qb = lax.shift_left(lax.shift_right_logical(j, SEGSH), 8)
            for (kin, _pi, _ko, _po, seghist, _of) in ios:
                d = lax.shift_right_logical(kin[sl], shv) & _DMASK
                cnt, last = plsc.scan_count(d)
                plsc.addupdate_scatter(seghist, [d + qb], cnt, mask=last)

        @plsc.parallel_loop(0, NC, unroll=2)
        def psum_body(c, ios=ios):
            sl = pl.ds(c * _NL, _NL)
            for (_ki, _pi, _ko, _po, seghist, offs) in ios:
                acc = jnp.zeros((_NL,), jnp.int32)
                for q in range(_NSEG):
                    offs[pl.ds(q * _RADIX + c * _NL, _NL)] = acc
                    acc = acc + seghist[pl.ds(q * _RADIX + c * _NL, _NL)]
                seghist[sl] = acc

        def gscan_body(c, carry):
            carry = list(carry)
            sl = pl.ds(c * _NL, _NL)
            for i, (_ki, _pi, _ko, _po, seghist, _of) in enumerate(ios):
                h = seghist[sl]
                cs = plsc.cumsum(h)
                seghist[sl] = cs - h + carry[i]
                carry[i] = carry[i] + cs[15]
            return tuple(carry)

        lax.fori_loop(0, NC, gscan_body, (jnp.int32(0),) * _NR)

        @plsc.parallel_loop(0, NC, unroll=2)
        def addg_body(c, ios=ios):
            sl = pl.ds(c * _NL, _NL)
            for (_ki, _pi, _ko, _po, seghist, offs) in ios:
                e = seghist[sl]
                for q in range(_NSEG):
                    osl = pl.ds(q * _RADIX + c * _NL, _NL)
                    offs[osl] = offs[osl] + e

        @plsc.parallel_loop(0, _NSEG, unroll=2)
        def perm_body(q, ios=ios, shv=shv):
            qb = lax.shift_left(q, 8)
            for (kin, pin, kout, pout, _sh, offs) in ios:
                for t in range(SEGV):
                    sl = pl.ds(q * SEGV * _NL + t * _NL, _NL)
                    k = kin[sl]
                    d = lax.shift_right_logical(k, shv) & _DMASK
                    dq = d + qb
                    cnt, last = plsc.scan_count(d)
                    pos = plsc.load_gather(offs, [dq]) + cnt - 1
                    plsc.store_scatter(kout, [pos], k)
                    plsc.store_scatter(pout, [pos], pin[sl])
                    plsc.store_scatter(offs, [dq], pos + 1, mask=last)

    ios_ab = [(r[1], r[2], r[3], r[4], r[5], r[6]) for r in rows]
    ios_ba = [(r[3], r[4], r[1], r[2], r[5], r[6]) for r in rows]

    def dp_body(dp, _):
        sh0 = dp * 16
        _one_pass(ios_ab, sh0)
        _one_pass(ios_ba, sh0 + 8)
        return 0

    lax.fori_loop(0, len(_SHIFTS) // 2, dp_body, 0)

    fin_a = len(_SHIFTS) % 2 == 0
    big = jnp.int32(S + 10)

    @plsc.parallel_loop(0, NV, unroll=_UNROLL)
    def lab_local(j):
        sl = pl.ds(j * _NL, _NL)
        p_vec = lax.iota(jnp.int32, _NL) + j * _NL
        for r in rows:
            kfin = r[1] if fin_a else r[3]
            abuf = r[3] if fin_a else r[1]
            uu = kfin[sl]
            flip = jnp.where(uu < 0, jnp.int32(-(2 ** 31)), jnp.int32(-1))
            v = plsc.bitcast(uu ^ flip, jnp.float32)
            x = jnp.clip(v + 0.5, 2.0, 4096.0)
            a = x.astype(jnp.int32) - p_vec
            a = jnp.where(p_vec >= 2, a, big)
            abuf[sl] = -plsc.cummax(-a)

    def lab_prefix(c, carry):
        carry = list(carry)
        tail_idx = lax.iota(jnp.int32, _NL) * _NL + (c * _NL * _NL + _NL - 1)
        dst_idx = lax.iota(jnp.int32, _NL) + (c * _NL + 1)
        for i, r in enumerate(rows):
            abuf = r[3] if fin_a else r[1]
            pbuf = r[4] if fin_a else r[2]
            mins = plsc.load_gather(abuf, [tail_idx])
            incl = jnp.minimum(-plsc.cummax(-mins), carry[i])
            plsc.store_scatter(pbuf, [dst_idx], incl)
            carry[i] = incl[15]
        return tuple(carry)

    lax.fori_loop(0, NV // _NL, lab_prefix, (big,) * _NR)

    @plsc.parallel_loop(0, NV, unroll=_UNROLL)
    def lab_emit(j):
        sl = pl.ds(j * _NL, _NL)
        p_vec = lax.iota(jnp.int32, _NL) + j * _NL
        for r in rows:
            abuf = r[3] if fin_a else r[1]
            pbuf = r[4] if fin_a else r[2]
            pfin = r[2] if fin_a else r[4]
            obuf = r[7]
            pre = plsc.load_gather(pbuf, [j + jnp.zeros((_NL,), jnp.int32)])
            pre = jnp.where(j >= 1, pre, big)
            g = jnp.minimum(abuf[sl], pre)
            lab = (p_vec + jnp.minimum(g, 1)).astype(jnp.float32)
            plsc.store_scatter(obuf, [pfin[sl]], lab)


def kernel(norms):
    B, S = norms.shape
    mesh = plsc.VectorSubcoreMesh(core_axis_name="c", subcore_axis_name="s")
    n_workers = mesh.num_cores * mesh.num_subcores
    rows_per_w = B // n_workers

    scratch = []
    for _ in range(_NR):
        scratch += [
            pltpu.VMEM((S,), jnp.float32),
            pltpu.VMEM((S,), jnp.int32),
            pltpu.VMEM((S,), jnp.int32),
            pltpu.VMEM((S,), jnp.int32),
            pltpu.VMEM((S,), jnp.int32),
            pltpu.VMEM((_NSEG * _RADIX,), jnp.int32),
            pltpu.VMEM((_NSEG * _RADIX,), jnp.int32),
            pltpu.VMEM((S,), jnp.float32),
        ]

    @functools.partial(
        pl.kernel,
        out_type=jax.ShapeDtypeStruct((B, S), jnp.float32),
        mesh=mesh,
        scratch_types=scratch,
        compiler_params=pltpu.CompilerParams(needs_layout_passes=False),
    )
    def sc_kernel(norms_hbm, out_hbm, *bufs):
        rows = [tuple(bufs[8 * i:8 * i + 8]) for i in range(_NR)]
        wid = lax.axis_index("s") * mesh.num_cores + lax.axis_index("c")
        row0 = wid * rows_per_w

        def row_body(r, _):
            row = row0 + r * _NR
            for i in range(_NR):
                pltpu.sync_copy(norms_hbm.at[row + i], rows[i][0])
            _rows_sort_and_label(rows, S)
            for i in range(_NR):
                pltpu.sync_copy(rows[i][7], out_hbm.at[row + i])
            return 0

        lax.fori_loop(0, rows_per_w // _NR, row_body, 0)

    return sc_kernel(norms)


# speedup vs baseline: 1.2293x; 1.2293x over previous
import functools

import jax
import jax.numpy as jnp
from jax import lax
from jax.experimental import pallas as pl
from jax.experimental.pallas import tpu as pltpu
from jax.experimental.pallas import tpu_sc as plsc

_NL = 16
_RADIX = 2048
_DMASK = _RADIX - 1
_SHIFTS = (0, 11, 22)
_NR = 4
_UNROLL = 4


def _rows_sort_and_label(rows, S):
    NV = S // _NL
    NH = _RADIX // _NL

    @plsc.parallel_loop(0, NV, unroll=_UNROLL)
    def key_init(j):
        sl = pl.ds(j * _NL, _NL)
        for (vbuf, key_a, pay_a, *_rest) in rows:
            b = plsc.bitcast(vbuf[sl], jnp.int32)
            flip = jnp.where(b < 0, jnp.int32(-1), jnp.int32(-(2 ** 31)))
            key_a[sl] = b ^ flip
            pay_a[sl] = lax.iota(jnp.int32, _NL) + j * _NL

    for pidx, sh in enumerate(_SHIFTS):
        if pidx % 2 == 0:
            ios = [(r[1], r[2], r[3], r[4], r[5], r[7], r[8]) for r in rows]
        else:
            ios = [(r[3], r[4], r[1], r[2], r[5], r[7], r[8]) for r in rows]

        @plsc.parallel_loop(0, NH, unroll=_UNROLL)
        def zero_body(c, ios=ios):
            sl = pl.ds(c * _NL, _NL)
            for io in ios:
                io[4][sl] = jnp.zeros((_NL,), jnp.int32)

        @plsc.parallel_loop(0, NV, unroll=_UNROLL)
        def hist_body(j, ios=ios, sh=sh):
            sl = pl.ds(j * _NL, _NL)
            for (kin, _pi, _ko, _po, hist, cbuf, lbuf) in ios:
                d = lax.shift_right_logical(kin[sl], sh) & _DMASK
                cnt, last = plsc.scan_count(d)
                cbuf[sl] = cnt - 1
                lbuf[sl] = jnp.where(last, jnp.int32(1), jnp.int32(0))
                plsc.addupdate_scatter(hist, [d], cnt, mask=last)

        def scan_body(cc, carry, ios=ios):
            carry = list(carry)
            for u in range(_UNROLL):
                sl = pl.ds((cc * _UNROLL + u) * _NL, _NL)
                for i, io in enumerate(ios):
                    hist = io[4]
                    h = hist[sl]
                    cs = plsc.cumsum(h)
                    hist[sl] = cs - h + carry[i]
                    carry[i] = carry[i] + cs[15]
            return tuple(carry)

        lax.fori_loop(0, NH // _UNROLL, scan_body, (jnp.int32(0),) * _NR)

        def perm_body(jj, _, ios=ios, sh=sh):
            for u in range(_UNROLL):
                sl = pl.ds((jj * _UNROLL + u) * _NL, _NL)
                for (kin, pin, kout, pout, hist, cbuf, lbuf) in ios:
                    k = kin[sl]
                    d = lax.shift_right_logical(k, sh) & _DMASK
                    last = lbuf[sl] == 1
                    pos = plsc.load_gather(hist, [d]) + cbuf[sl]
                    plsc.store_scatter(kout, [pos], k)
                    plsc.store_scatter(pout, [pos], pin[sl])
                    plsc.store_scatter(hist, [d], pos + 1, mask=last)
            return 0

        lax.fori_loop(0, NV // _UNROLL, perm_body, 0)

    fin_a = len(_SHIFTS) % 2 == 0
    big = jnp.int32(S + 10)

    @plsc.parallel_loop(0, NV, unroll=_UNROLL)
    def lab_local(j):
        sl = pl.ds(j * _NL, _NL)
        p_vec = lax.iota(jnp.int32, _NL) + j * _NL
        for r in rows:
            kfin = r[1] if fin_a else r[3]
            abuf = r[3] if fin_a else r[1]
            uu = kfin[sl]
            flip = jnp.where(uu < 0, jnp.int32(-(2 ** 31)), jnp.int32(-1))
            v = plsc.bitcast(uu ^ flip, jnp.float32)
            x = jnp.clip(v + 0.5, 2.0, 4096.0)
            a = x.astype(jnp.int32) - p_vec
            a = jnp.where(p_vec >= 2, a, big)
            abuf[sl] = -plsc.cummax(-a)

    def lab_prefix(c, carry):
        carry = list(carry)
        tail_idx = lax.iota(jnp.int32, _NL) * _NL + (c * _NL * _NL + _NL - 1)
        dst_idx = lax.iota(jnp.int32, _NL) + (c * _NL + 1)
        for i, r in enumerate(rows):
            abuf = r[3] if fin_a else r[1]
            pbuf = r[4] if fin_a else r[2]
            mins = plsc.load_gather(abuf, [tail_idx])
            incl = jnp.minimum(-plsc.cummax(-mins), carry[i])
            plsc.store_scatter(pbuf, [dst_idx], incl)
            carry[i] = incl[15]
        return tuple(carry)

    lax.fori_loop(0, NV // _NL, lab_prefix, (big,) * _NR)

    @plsc.parallel_loop(0, NV, unroll=_UNROLL)
    def lab_emit(j):
        sl = pl.ds(j * _NL, _NL)
        p_vec = lax.iota(jnp.int32, _NL) + j * _NL
        for r in rows:
            abuf = r[3] if fin_a else r[1]
            pbuf = r[4] if fin_a else r[2]
            pfin = r[2] if fin_a else r[4]
            obuf = r[6]
            pre = plsc.load_gather(pbuf, [jnp.where(j >= 1, j, 0)
                                          + jnp.zeros((_NL,), jnp.int32)])
            pre = jnp.where(j >= 1, pre, big)
            g = jnp.minimum(abuf[sl], pre)
            lab = (p_vec + jnp.minimum(g, 1)).astype(jnp.float32)
            plsc.store_scatter(obuf, [pfin[sl]], lab)


def kernel(norms):
    B, S = norms.shape
    mesh = plsc.VectorSubcoreMesh(core_axis_name="c", subcore_axis_name="s")
    n_workers = mesh.num_cores * mesh.num_subcores
    rows_per_w = B // n_workers

    scratch = []
    for _ in range(_NR):
        scratch += [
            pltpu.VMEM((S,), jnp.float32),
            pltpu.VMEM((S,), jnp.int32),
            pltpu.VMEM((S,), jnp.int32),
            pltpu.VMEM((S,), jnp.int32),
            pltpu.VMEM((S,), jnp.int32),
            pltpu.VMEM((_RADIX,), jnp.int32),
            pltpu.VMEM((S,), jnp.float32),
            pltpu.VMEM((S,), jnp.int32),
            pltpu.VMEM((S,), jnp.int32),
        ]

    @functools.partial(
        pl.kernel,
        out_type=jax.ShapeDtypeStruct((B, S), jnp.float32),
        mesh=mesh,
        scratch_types=scratch,
        compiler_params=pltpu.CompilerParams(needs_layout_passes=False),
    )
    def sc_kernel(norms_hbm, out_hbm, *bufs):
        rows = [tuple(bufs[9 * i:9 * i + 9]) for i in range(_NR)]
        wid = lax.axis_index("s") * mesh.num_cores + lax.axis_index("c")
        row0 = wid * rows_per_w

        def row_body(r, _):
            row = row0 + r * _NR
            for i in range(_NR):
                pltpu.sync_copy(norms_hbm.at[row + i], rows[i][0])
            _rows_sort_and_label(rows, S)
            for i in range(_NR):
                pltpu.sync_copy(rows[i][6], out_hbm.at[row + i])
            return 0

        lax.fori_loop(0, rows_per_w // _NR, row_body, 0)

    return sc_kernel(norms)
